# baseline (speedup 1.0000x reference)
import jax
import jax.numpy as jnp
from jax import lax
from jax.experimental import pallas as pl
from jax.experimental.pallas import tpu as pltpu
from jax.experimental.pallas import tpu_sc as plsc

N = 10000
NP = 10240
E = 160000
EPAD = 163840
IN_CH = 256
H = 512
CN = 16
NC = 10

CAP32 = 5632
CAP64 = 2944

_f32 = jnp.float32
_i32 = jnp.int32

_BN = 512
_GRID = NP // _BN


_NLP = pltpu.CompilerParams(needs_layout_passes=False)


def _deg_body(dst_hbm, out_hbm, acc, dstv):
    cc = lax.axis_index("c")
    s = lax.axis_index("s")
    wid = cc * 16 + s
    rows = NP // 32
    lo = wid * rows
    io16 = lax.iota(_i32, 16)
    ones = jnp.ones((16,), _f32)

    def zbody(i, carry):
        acc[i, pl.ds(0, 16)] = jnp.zeros((16,), _f32)
        return carry
    lax.fori_loop(0, rows + 8, zbody, 0)

    base = wid * CAP32
    nb = CAP32 // 128

    def bat(k, carry):
        eb = base + k * 128
        pltpu.sync_copy(dst_hbm.at[pl.ds(eb, 128)], dstv)
        lvs = []
        for cj in range(8):
            lv = dstv[pl.ds(cj * 16, 16)] - lo
            ok = (lv >= 0) & (lv < rows)
            lvs.append(jnp.where(ok, lv, rows))
        for e in range(128):
            le = lvs[e // 16].at[jnp.full((16,), e % 16, _i32)].get(
                mode="promise_in_bounds")
            plsc.addupdate_scatter(acc, [le, io16], ones)
        return carry
    lax.fori_loop(0, nb, bat, 0)
    pltpu.sync_copy(acc.at[pl.ds(0, rows)], out_hbm.at[pl.ds(lo, rows)])


def _make_deg():
    mesh = plsc.VectorSubcoreMesh(core_axis_name="c", subcore_axis_name="s")
    return pl.kernel(
        _deg_body,
        out_type=jax.ShapeDtypeStruct((NP, 16), _f32),
        mesh=mesh,
        compiler_params=_NLP,
        scratch_types=[
            pltpu.VMEM((NP // 32 + 8, 16), _f32),
            pltpu.VMEM((128,), _i32),
        ],
    )


def _make_agg(C, nranges, cap, B):
    rows = NP // nranges
    passes = nranges // 32
    nb = cap // B
    assert cap % B == 0 and B % 16 == 0

    mesh = plsc.VectorSubcoreMesh(core_axis_name="c", subcore_axis_name="s")

    def body(y_hbm, srcs_hbm, dsts_hbm, out_hbm, acc, srcv, dstv, rowsbuf, sem):
        cc = lax.axis_index("c")
        s = lax.axis_index("s")
        wid = cc * 16 + s
        io16 = lax.iota(_i32, 16)
        for p in range(passes):
            rid = p * 32 + wid
            lo = rid * rows

            def zbody(i, carry):
                for j in range(C // 16):
                    acc[i, pl.ds(j * 16, 16)] = jnp.zeros((16,), _f32)
                return carry
            lax.fori_loop(0, rows + 8, zbody, 0)

            base = rid * cap

            def bat(k, carry):
                eb = base + k * B
                pltpu.sync_copy(srcs_hbm.at[pl.ds(eb, B)], srcv)
                cp = pltpu.async_copy(y_hbm.at[srcv], rowsbuf, sem)
                pltpu.sync_copy(dsts_hbm.at[pl.ds(eb, B)], dstv)
                lvs = []
                for cj in range(B // 16):
                    lv = dstv[pl.ds(cj * 16, 16)] - lo
                    ok = (lv >= 0) & (lv < rows)
                    lvs.append(jnp.where(ok, lv, rows))
                cp.wait()
                for e in range(B):
                    le = lvs[e // 16].at[jnp.full((16,), e % 16, _i32)].get(
                        mode="promise_in_bounds")
                    for j in range(C // 16):
                        plsc.addupdate_scatter(
                            acc, [le, io16 + (j * 16)],
                            rowsbuf[e, pl.ds(j * 16, 16)])
                return carry

            lax.fori_loop(0, nb, bat, 0)
            pltpu.sync_copy(acc.at[pl.ds(0, rows)],
                            out_hbm.at[pl.ds(lo, rows)])

    return pl.kernel(
        body,
        out_type=jax.ShapeDtypeStruct((NP, C), _f32),
        mesh=mesh,
        compiler_params=_NLP,
        scratch_types=[
            pltpu.VMEM((rows + 8, C), _f32),
            pltpu.VMEM((B,), _i32),
            pltpu.VMEM((B,), _i32),
            pltpu.VMEM((B, C), _f32),
            pltpu.SemaphoreType.DMA,
        ],
    )


_PREC = lax.Precision.HIGHEST
_bf16 = jnp.bfloat16


def _bdot(a, b):
    return jnp.dot(a.astype(_bf16), b.astype(_bf16),
                   preferred_element_type=_f32)


def _bdot_t(a, b):
    dn = (((1,), (1,)), ((), ()))
    return lax.dot_general(a.astype(_bf16), b.astype(_bf16), dn,
                           preferred_element_type=_f32)


def _prep_body(degw_ref, x_ref, W1_ref, y0h_ref, dinv_ref, deg_ref):
    dg = degw_ref[:, 0:1]
    dinv = 1.0 / jnp.sqrt(dg + 1.0)
    dinv_ref[...] = dinv
    deg_ref[...] = dg
    h = _bdot(x_ref[...], W1_ref[...])
    y0h_ref[...] = h * dinv


def _tc_prep(degw, xq, W1):
    return pl.pallas_call(
        _prep_body,
        grid=(_GRID,),
        in_specs=[
            pl.BlockSpec((_BN, 16), lambda i: (i, 0)),
            pl.BlockSpec((_BN, IN_CH), lambda i: (i, 0)),
            pl.BlockSpec((IN_CH, H), lambda i: (0, 0)),
        ],
        out_specs=[
            pl.BlockSpec((_BN, H), lambda i: (i, 0)),
            pl.BlockSpec((_BN, 1), lambda i: (i, 0)),
            pl.BlockSpec((_BN, 1), lambda i: (i, 0)),
        ],
        out_shape=[
            jax.ShapeDtypeStruct((NP, H), _f32),
            jax.ShapeDtypeStruct((NP, 1), _f32),
            jax.ShapeDtypeStruct((NP, 1), _f32),
        ],
    )(degw, xq, W1)


def _mm2_body(S0_ref, y0h_ref, dinv_ref, b1_ref, W2_ref, y1g_ref):
    dinv = dinv_ref[...]
    h1 = jnp.maximum((S0_ref[...] + y0h_ref[...]) * dinv + b1_ref[...], 0.0)
    g = _bdot(h1, W2_ref[...])
    y1g_ref[...] = g * dinv


def _tc_mm2(S0, y0h, dinvc, b1r, W2):
    return pl.pallas_call(
        _mm2_body,
        grid=(_GRID,),
        in_specs=[
            pl.BlockSpec((_BN, H), lambda i: (i, 0)),
            pl.BlockSpec((_BN, H), lambda i: (i, 0)),
            pl.BlockSpec((_BN, 1), lambda i: (i, 0)),
            pl.BlockSpec((1, H), lambda i: (0, 0)),
            pl.BlockSpec((H, H), lambda i: (0, 0)),
        ],
        out_specs=pl.BlockSpec((_BN, H), lambda i: (i, 0)),
        out_shape=jax.ShapeDtypeStruct((NP, H), _f32),
    )(S0, y0h, dinvc, b1r, W2)


def _h2_body(S1_ref, y1g_ref, dinv_ref, b2_ref, h2_ref):
    h2_ref[...] = ((S1_ref[...] + y1g_ref[...]) * dinv_ref[...]
                   + b2_ref[...])


def _tc_h2(S1, y1g, dinvc, b2r):
    return pl.pallas_call(
        _h2_body,
        grid=(_GRID,),
        in_specs=[
            pl.BlockSpec((_BN, H), lambda i: (i, 0)),
            pl.BlockSpec((_BN, H), lambda i: (i, 0)),
            pl.BlockSpec((_BN, 1), lambda i: (i, 0)),
            pl.BlockSpec((1, H), lambda i: (0, 0)),
        ],
        out_specs=pl.BlockSpec((_BN, H), lambda i: (i, 0)),
        out_shape=jax.ShapeDtypeStruct((NP, H), _f32),
    )(S1, y1g, dinvc, b2r)


def _final_body(h2_ref, S2_ref, deg_ref, Wi_ref, Wbm_ref, feat_ref, out_ref):
    h2 = h2_ref[...]
    neigh = S2_ref[...] / jnp.maximum(deg_ref[...], 1.0)
    feat = jnp.concatenate([h2, neigh], axis=1)
    feat_ref[...] = feat
    logits = _bdot_t(feat, Wi_ref[...])
    mx = jnp.max(logits, axis=1, keepdims=True)
    io = lax.broadcasted_iota(_i32, (_BN, CN), 1)
    idx = jnp.min(jnp.where(logits == mx, io, CN), axis=1, keepdims=True)
    allout = _bdot_t(feat, Wbm_ref[...])
    oh = jnp.where(io == idx, 1.0, 0.0)
    rep_r = lax.broadcasted_iota(_i32, (CN, CN * NC), 0)
    rep_c = lax.broadcasted_iota(_i32, (CN, CN * NC), 1)
    REP = jnp.where(rep_c // NC == rep_r, 1.0, 0.0)
    g_r = lax.broadcasted_iota(_i32, (CN * NC, NC), 0)
    g_c = lax.broadcasted_iota(_i32, (CN * NC, NC), 1)
    G = jnp.where(g_r % NC == g_c, 1.0, 0.0)
    ohrep = jnp.dot(oh, REP, preferred_element_type=_f32, precision=_PREC)
    out_ref[...] = jnp.dot(allout * ohrep, G,
                           preferred_element_type=_f32, precision=_PREC)


def _tc_final(h2, S2, degc, Wi, Wbm):
    return pl.pallas_call(
        _final_body,
        grid=(_GRID,),
        in_specs=[
            pl.BlockSpec((_BN, H), lambda i: (i, 0)),
            pl.BlockSpec((_BN, H), lambda i: (i, 0)),
            pl.BlockSpec((_BN, 1), lambda i: (i, 0)),
            pl.BlockSpec((CN, 2 * H), lambda i: (0, 0)),
            pl.BlockSpec((CN * NC, 2 * H), lambda i: (0, 0)),
        ],
        out_specs=[
            pl.BlockSpec((_BN, 2 * H), lambda i: (i, 0)),
            pl.BlockSpec((_BN, NC), lambda i: (i, 0)),
        ],
        out_shape=[
            jax.ShapeDtypeStruct((NP, 2 * H), _f32),
            jax.ShapeDtypeStruct((NP, NC), _f32),
        ],
    )(h2, S2, degc, Wi, Wbm)


_deg_kernel = _make_deg()
_agg_h = _make_agg(H, 64, CAP64, 32)


def _slabs(srcs, dsts, nranges, cap):
    rows = NP // nranges
    i = jnp.arange(EPAD, dtype=_i32)
    b = jnp.searchsorted(dsts, jnp.arange(nranges + 1, dtype=_i32) * rows,
                         ).astype(_i32)
    r_real = jnp.clip(dsts // rows, 0, nranges - 1)
    pos_real = r_real * cap + (i - b[r_real])
    j = i - E
    rp = j % nranges
    pos_pad = rp * cap + (b[rp + 1] - b[rp]) + j // nranges
    pos = jnp.where(i < E, pos_real, pos_pad)
    ssrc = jnp.full((nranges * cap,), NP - 1, _i32).at[pos].set(srcs)
    sdst = jnp.full((nranges * cap,), NP, _i32).at[pos].set(
        jnp.where(i < E, dsts, NP))
    return ssrc, sdst


def kernel(x, edge_index, W1, b1, W2, b2, Wi, Wb):
    src = edge_index[0].astype(_i32)
    dst = edge_index[1].astype(_i32)
    xq = jnp.pad(x, ((0, NP - N), (0, 0)))

    padi = jnp.full((EPAD - E,), NP, _i32)
    dstp = jnp.concatenate([dst, padi])
    srcp = jnp.concatenate([src, jnp.full((EPAD - E,), NP - 1, _i32)])
    order = jnp.argsort(dstp)
    dsts = dstp[order]
    srcs = srcp[order]
    src32, dst32 = _slabs(srcs, dsts, 32, CAP32)
    src64, dst64 = _slabs(srcs, dsts, 64, CAP64)

    degw = _deg_kernel(dst32)

    y0h, dinvc, degc = _tc_prep(degw, xq, W1)
    S0 = _agg_h(y0h, src64, dst64)
    y1g = _tc_mm2(S0, y0h, dinvc, b1.reshape(1, H), W2)
    S1 = _agg_h(y1g, src64, dst64)
    h2 = _tc_h2(S1, y1g, dinvc, b2.reshape(1, H))
    S2 = _agg_h(h2, src64, dst64)
    feat, out = _tc_final(h2, S2, degc, Wi, Wb.reshape(CN * NC, 2 * H))
    return out[:N], feat[:N]


# speedup vs baseline: 1.1327x; 1.1327x over previous
import jax
import jax.numpy as jnp
from jax import lax
from jax.experimental import pallas as pl
from jax.experimental.pallas import tpu as pltpu
from jax.experimental.pallas import tpu_sc as plsc

N = 10000
NP = 10240
E = 160000
EPAD = 163840
IN_CH = 256
H = 512
CN = 16
NC = 10

CAP32 = 5632
CAP64 = 2944

_f32 = jnp.float32
_i32 = jnp.int32

_BN = 512
_GRID = NP // _BN


_NLP = pltpu.CompilerParams(needs_layout_passes=False)


def _deg_body(dst_hbm, out_hbm, acc, dstv):
    cc = lax.axis_index("c")
    s = lax.axis_index("s")
    wid = cc * 16 + s
    rows = NP // 32
    lo = wid * rows
    io16 = lax.iota(_i32, 16)
    ones = jnp.ones((16,), _f32)

    def zbody(i, carry):
        acc[i, pl.ds(0, 16)] = jnp.zeros((16,), _f32)
        return carry
    lax.fori_loop(0, rows + 8, zbody, 0)

    base = wid * CAP32
    nb = CAP32 // 128

    def bat(k, carry):
        eb = base + k * 128
        pltpu.sync_copy(dst_hbm.at[pl.ds(eb, 128)], dstv)
        lvs = []
        for cj in range(8):
            lv = dstv[pl.ds(cj * 16, 16)] - lo
            ok = (lv >= 0) & (lv < rows)
            lvs.append(jnp.where(ok, lv, rows))
        for e in range(128):
            le = lvs[e // 16].at[jnp.full((16,), e % 16, _i32)].get(
                mode="promise_in_bounds")
            plsc.addupdate_scatter(acc, [le, io16], ones)
        return carry
    lax.fori_loop(0, nb, bat, 0)
    pltpu.sync_copy(acc.at[pl.ds(0, rows)], out_hbm.at[pl.ds(lo, rows)])


def _make_deg():
    mesh = plsc.VectorSubcoreMesh(core_axis_name="c", subcore_axis_name="s")
    return pl.kernel(
        _deg_body,
        out_type=jax.ShapeDtypeStruct((NP, 16), _f32),
        mesh=mesh,
        compiler_params=_NLP,
        scratch_types=[
            pltpu.VMEM((NP // 32 + 8, 16), _f32),
            pltpu.VMEM((128,), _i32),
        ],
    )


def _make_agg(C, nranges, cap, B):
    rows = NP // nranges
    passes = nranges // 32
    nb = cap // B
    assert cap % B == 0 and B % 16 == 0

    mesh = plsc.VectorSubcoreMesh(core_axis_name="c", subcore_axis_name="s")

    def body(y_hbm, srcs_hbm, dsts_hbm, out_hbm,
             acc, srcv0, dstv0, buf0, srcv1, dstv1, buf1, sem0, sem1):
        cc = lax.axis_index("c")
        s = lax.axis_index("s")
        wid = cc * 16 + s
        io16 = lax.iota(_i32, 16)
        bufs = ((srcv0, dstv0, buf0, sem0), (srcv1, dstv1, buf1, sem1))

        def compute(dstv, rowsbuf, lo):
            def ebody(e, carry):
                es = jnp.full((16,), e, _i32)
                dvec = plsc.load_gather(dstv, [es])
                lv = dvec - lo
                ok = (lv >= 0) & (lv < rows)
                le = jnp.where(ok, lv, rows)
                for g in range(C // 256):
                    vals = [rowsbuf[e, pl.ds((g * 16 + j) * 16, 16)]
                            for j in range(16)]
                    for j in range(16):
                        plsc.addupdate_scatter(
                            acc, [le, io16 + ((g * 16 + j) * 16)], vals[j])
                return carry
            lax.fori_loop(0, B, ebody, 0)

        def fetch(k, base, srcv, dstv, rowsbuf, sem):
            eb = base + k * B
            pltpu.sync_copy(srcs_hbm.at[pl.ds(eb, B)], srcv)
            pltpu.sync_copy(dsts_hbm.at[pl.ds(eb, B)], dstv)
            pltpu.async_copy(y_hbm.at[srcv], rowsbuf, sem)

        def gwait(srcv, rowsbuf, sem):
            pltpu.make_async_copy(y_hbm.at[srcv], rowsbuf, sem).wait()

        for p in range(passes):
            rid = p * 32 + wid
            lo = rid * rows

            def zbody(i, carry):
                for j in range(C // 16):
                    acc[i, pl.ds(j * 16, 16)] = jnp.zeros((16,), _f32)
                return carry
            lax.fori_loop(0, rows + 8, zbody, 0)

            base = rid * cap
            fetch(0, base, srcv0, dstv0, buf0, sem0)

            def bat(m, carry):
                fetch(2 * m + 1, base, srcv1, dstv1, buf1, sem1)
                gwait(srcv0, buf0, sem0)
                compute(dstv0, buf0, lo)
                fetch(2 * m + 2, base, srcv0, dstv0, buf0, sem0)
                gwait(srcv1, buf1, sem1)
                compute(dstv1, buf1, lo)
                return carry

            lax.fori_loop(0, nb // 2 - 1, bat, 0)
            fetch(nb - 1, base, srcv1, dstv1, buf1, sem1)
            gwait(srcv0, buf0, sem0)
            compute(dstv0, buf0, lo)
            gwait(srcv1, buf1, sem1)
            compute(dstv1, buf1, lo)

            pltpu.sync_copy(acc.at[pl.ds(0, rows)],
                            out_hbm.at[pl.ds(lo, rows)])

    return pl.kernel(
        body,
        out_type=jax.ShapeDtypeStruct((NP, C), _f32),
        mesh=mesh,
        compiler_params=_NLP,
        scratch_types=[
            pltpu.VMEM((rows + 8, C), _f32),
            pltpu.VMEM((B,), _i32),
            pltpu.VMEM((B,), _i32),
            pltpu.VMEM((B, C), _f32),
            pltpu.VMEM((B,), _i32),
            pltpu.VMEM((B,), _i32),
            pltpu.VMEM((B, C), _f32),
            pltpu.SemaphoreType.DMA,
            pltpu.SemaphoreType.DMA,
        ],
    )


_PREC = lax.Precision.HIGHEST
_bf16 = jnp.bfloat16


def _bdot(a, b):
    return jnp.dot(a.astype(_bf16), b.astype(_bf16),
                   preferred_element_type=_f32)


def _bdot_t(a, b):
    dn = (((1,), (1,)), ((), ()))
    return lax.dot_general(a.astype(_bf16), b.astype(_bf16), dn,
                           preferred_element_type=_f32)


def _prep_body(degw_ref, x_ref, W1_ref, y0h_ref, dinv_ref, deg_ref):
    dg = degw_ref[:, 0:1]
    dinv = 1.0 / jnp.sqrt(dg + 1.0)
    dinv_ref[...] = dinv
    deg_ref[...] = dg
    h = _bdot(x_ref[...], W1_ref[...])
    y0h_ref[...] = h * dinv


def _tc_prep(degw, xq, W1):
    return pl.pallas_call(
        _prep_body,
        grid=(_GRID,),
        in_specs=[
            pl.BlockSpec((_BN, 16), lambda i: (i, 0)),
            pl.BlockSpec((_BN, IN_CH), lambda i: (i, 0)),
            pl.BlockSpec((IN_CH, H), lambda i: (0, 0)),
        ],
        out_specs=[
            pl.BlockSpec((_BN, H), lambda i: (i, 0)),
            pl.BlockSpec((_BN, 1), lambda i: (i, 0)),
            pl.BlockSpec((_BN, 1), lambda i: (i, 0)),
        ],
        out_shape=[
            jax.ShapeDtypeStruct((NP, H), _f32),
            jax.ShapeDtypeStruct((NP, 1), _f32),
            jax.ShapeDtypeStruct((NP, 1), _f32),
        ],
    )(degw, xq, W1)


def _mm2_body(S0_ref, y0h_ref, dinv_ref, b1_ref, W2_ref, y1g_ref):
    dinv = dinv_ref[...]
    h1 = jnp.maximum((S0_ref[...] + y0h_ref[...]) * dinv + b1_ref[...], 0.0)
    g = _bdot(h1, W2_ref[...])
    y1g_ref[...] = g * dinv


def _tc_mm2(S0, y0h, dinvc, b1r, W2):
    return pl.pallas_call(
        _mm2_body,
        grid=(_GRID,),
        in_specs=[
            pl.BlockSpec((_BN, H), lambda i: (i, 0)),
            pl.BlockSpec((_BN, H), lambda i: (i, 0)),
            pl.BlockSpec((_BN, 1), lambda i: (i, 0)),
            pl.BlockSpec((1, H), lambda i: (0, 0)),
            pl.BlockSpec((H, H), lambda i: (0, 0)),
        ],
        out_specs=pl.BlockSpec((_BN, H), lambda i: (i, 0)),
        out_shape=jax.ShapeDtypeStruct((NP, H), _f32),
    )(S0, y0h, dinvc, b1r, W2)


def _h2_body(S1_ref, y1g_ref, dinv_ref, b2_ref, h2_ref):
    h2_ref[...] = ((S1_ref[...] + y1g_ref[...]) * dinv_ref[...]
                   + b2_ref[...])


def _tc_h2(S1, y1g, dinvc, b2r):
    return pl.pallas_call(
        _h2_body,
        grid=(_GRID,),
        in_specs=[
            pl.BlockSpec((_BN, H), lambda i: (i, 0)),
            pl.BlockSpec((_BN, H), lambda i: (i, 0)),
            pl.BlockSpec((_BN, 1), lambda i: (i, 0)),
            pl.BlockSpec((1, H), lambda i: (0, 0)),
        ],
        out_specs=pl.BlockSpec((_BN, H), lambda i: (i, 0)),
        out_shape=jax.ShapeDtypeStruct((NP, H), _f32),
    )(S1, y1g, dinvc, b2r)


def _final_body(h2_ref, S2_ref, deg_ref, Wi_ref, Wbm_ref, feat_ref, out_ref):
    h2 = h2_ref[...]
    neigh = S2_ref[...] / jnp.maximum(deg_ref[...], 1.0)
    feat = jnp.concatenate([h2, neigh], axis=1)
    feat_ref[...] = feat
    logits = _bdot_t(feat, Wi_ref[...])
    mx = jnp.max(logits, axis=1, keepdims=True)
    io = lax.broadcasted_iota(_i32, (_BN, CN), 1)
    idx = jnp.min(jnp.where(logits == mx, io, CN), axis=1, keepdims=True)
    allout = _bdot_t(feat, Wbm_ref[...])
    oh = jnp.where(io == idx, 1.0, 0.0)
    rep_r = lax.broadcasted_iota(_i32, (CN, CN * NC), 0)
    rep_c = lax.broadcasted_iota(_i32, (CN, CN * NC), 1)
    REP = jnp.where(rep_c // NC == rep_r, 1.0, 0.0)
    g_r = lax.broadcasted_iota(_i32, (CN * NC, NC), 0)
    g_c = lax.broadcasted_iota(_i32, (CN * NC, NC), 1)
    G = jnp.where(g_r % NC == g_c, 1.0, 0.0)
    ohrep = jnp.dot(oh, REP, preferred_element_type=_f32, precision=_PREC)
    out_ref[...] = jnp.dot(allout * ohrep, G,
                           preferred_element_type=_f32, precision=_PREC)


def _tc_final(h2, S2, degc, Wi, Wbm):
    return pl.pallas_call(
        _final_body,
        grid=(_GRID,),
        in_specs=[
            pl.BlockSpec((_BN, H), lambda i: (i, 0)),
            pl.BlockSpec((_BN, H), lambda i: (i, 0)),
            pl.BlockSpec((_BN, 1), lambda i: (i, 0)),
            pl.BlockSpec((CN, 2 * H), lambda i: (0, 0)),
            pl.BlockSpec((CN * NC, 2 * H), lambda i: (0, 0)),
        ],
        out_specs=[
            pl.BlockSpec((_BN, 2 * H), lambda i: (i, 0)),
            pl.BlockSpec((_BN, NC), lambda i: (i, 0)),
        ],
        out_shape=[
            jax.ShapeDtypeStruct((NP, 2 * H), _f32),
            jax.ShapeDtypeStruct((NP, NC), _f32),
        ],
    )(h2, S2, degc, Wi, Wbm)


_deg_kernel = _make_deg()
_agg_h = _make_agg(H, 64, CAP64, 32)


def _slabs(srcs, dsts, nranges, cap):
    rows = NP // nranges
    i = jnp.arange(EPAD, dtype=_i32)
    b = jnp.searchsorted(dsts, jnp.arange(nranges + 1, dtype=_i32) * rows,
                         ).astype(_i32)
    r_real = jnp.clip(dsts // rows, 0, nranges - 1)
    pos_real = r_real * cap + (i - b[r_real])
    j = i - E
    rp = j % nranges
    pos_pad = rp * cap + (b[rp + 1] - b[rp]) + j // nranges
    pos = jnp.where(i < E, pos_real, pos_pad)
    ssrc = jnp.full((nranges * cap,), NP - 1, _i32).at[pos].set(srcs)
    sdst = jnp.full((nranges * cap,), NP, _i32).at[pos].set(
        jnp.where(i < E, dsts, NP))
    return ssrc, sdst


def kernel(x, edge_index, W1, b1, W2, b2, Wi, Wb):
    src = edge_index[0].astype(_i32)
    dst = edge_index[1].astype(_i32)
    xq = jnp.pad(x, ((0, NP - N), (0, 0)))

    padi = jnp.full((EPAD - E,), NP, _i32)
    dstp = jnp.concatenate([dst, padi])
    srcp = jnp.concatenate([src, jnp.full((EPAD - E,), NP - 1, _i32)])
    order = jnp.argsort(dstp)
    dsts = dstp[order]
    srcs = srcp[order]
    src32, dst32 = _slabs(srcs, dsts, 32, CAP32)
    src64, dst64 = _slabs(srcs, dsts, 64, CAP64)

    degw = _deg_kernel(dst32)

    y0h, dinvc, degc = _tc_prep(degw, xq, W1)
    S0 = _agg_h(y0h, src64, dst64)
    y1g = _tc_mm2(S0, y0h, dinvc, b1.reshape(1, H), W2)
    S1 = _agg_h(y1g, src64, dst64)
    h2 = _tc_h2(S1, y1g, dinvc, b2.reshape(1, H))
    S2 = _agg_h(h2, src64, dst64)
    feat, out = _tc_final(h2, S2, degc, Wi, Wb.reshape(CN * NC, 2 * H))
    return out[:N], feat[:N]
